# baseline (speedup 1.0000x reference)
import functools

import jax
import jax.numpy as jnp
from jax import lax
from jax.experimental import pallas as pl
from jax.experimental.pallas import tpu as pltpu
from jax.experimental.pallas import tpu_sc as plsc

NS = 16
NC = 2
CHUNK = 128
EPW = 5120
HDR = 128
CAPD = 5248
STR = HDR + CAPD
BN = 1000
SHIFT = 14


def _z_body(x_ref, w_ref, o_ref):
    o_ref[...] = jnp.dot(x_ref[...], w_ref[0], preferred_element_type=jnp.float32)


def _z_table(x, rw, N, D, R, H):
    nb = N // BN
    return pl.pallas_call(
        _z_body,
        grid=(R, NC, nb),
        in_specs=[
            pl.BlockSpec((BN, D), lambda r, c, n: (n, 0)),
            pl.BlockSpec((1, D, H), lambda r, c, n: (r, 0, c)),
        ],
        out_specs=pl.BlockSpec(
            (BN, H), lambda r, c, n, _nb=nb, _R=R: (c * _R * _nb + r * _nb + n, 0)),
        out_shape=jax.ShapeDtypeStruct((NC * R * N, H), jnp.float32),
    )(x, rw)


def _ep_body(cd_ref, x_ref, d_ref, nm0_ref, nm1_ref, ws_ref, b_ref, o_ref):
    cd = cd_ref[0, 0]
    tm = (d_ref[...] == cd - 1).astype(jnp.float32)
    nm = jnp.concatenate([nm0_ref[0], nm1_ref[0]], axis=-1) * tm
    acc = x_ref[...] + jnp.dot(x_ref[...], ws_ref[...],
                               preferred_element_type=jnp.float32)
    o_ref[...] = jnp.maximum(acc + b_ref[...] + nm, 0.0)


def _epilogue(cd11, x, dist2d, nm, ws, bias2d, N, D, O, H):
    nb = N // BN
    return pl.pallas_call(
        _ep_body,
        grid=(nb,),
        in_specs=[
            pl.BlockSpec((1, 1), lambda n: (0, 0)),
            pl.BlockSpec((BN, D), lambda n: (n, 0)),
            pl.BlockSpec((BN, 1), lambda n: (n, 0)),
            pl.BlockSpec((1, BN, H), lambda n: (0, n, 0)),
            pl.BlockSpec((1, BN, H), lambda n: (1, n, 0)),
            pl.BlockSpec((D, O), lambda n: (0, 0)),
            pl.BlockSpec((1, O), lambda n: (0, 0)),
        ],
        out_specs=pl.BlockSpec((BN, O), lambda n: (n, 0)),
        out_shape=jax.ShapeDtypeStruct((N, O), jnp.float32),
    )(cd11, x, dist2d, nm, nm, ws, bias2d)


def _make_filter(N, E):
    mesh = plsc.VectorSubcoreMesh(
        core_axis_name="c", subcore_axis_name="s", num_cores=NC, num_subcores=NS)

    @functools.partial(
        pl.kernel,
        out_type=jax.ShapeDtypeStruct((NC * NS * STR,), jnp.int32),
        mesh=mesh,
        compiler_params=pltpu.CompilerParams(needs_layout_passes=False),
        scratch_types=[
            pltpu.VMEM((16,), jnp.int32),
            pltpu.VMEM((N,), jnp.int32),
            pltpu.VMEM((EPW,), jnp.int32),
            pltpu.VMEM((EPW,), jnp.int32),
            pltpu.VMEM((EPW,), jnp.int32),
            pltpu.VMEM((STR,), jnp.int32),
            pltpu.SemaphoreType.DMA,
            pltpu.SemaphoreType.DMA,
            pltpu.SemaphoreType.DMA,
        ],
    )
    def flt(cd_hbm, src_hbm, dst_hbm, typ_hbm, dist_hbm, pk_hbm,
            cd_v, dist_v, src_v, dst_v, typ_v, pk_v, sm0, sm1, sm2):
        c = lax.axis_index("c")
        s = lax.axis_index("s")
        g = c * NS + s
        start = lax.min(g * EPW, jnp.int32(E - EPW))
        skip = g * EPW - start
        pltpu.async_copy(src_hbm.at[pl.ds(start, EPW)], src_v, sm0)
        pltpu.async_copy(dst_hbm.at[pl.ds(start, EPW)], dst_v, sm1)
        pltpu.async_copy(typ_hbm.at[pl.ds(start, EPW)], typ_v, sm2)
        pltpu.sync_copy(cd_hbm, cd_v)
        pltpu.sync_copy(dist_hbm, dist_v)
        pltpu.make_async_copy(src_hbm.at[pl.ds(start, EPW)], src_v, sm0).wait()
        pltpu.make_async_copy(dst_hbm.at[pl.ds(start, EPW)], dst_v, sm1).wait()
        pltpu.make_async_copy(typ_hbm.at[pl.ds(start, EPW)], typ_v, sm2).wait()

        cdvec = cd_v[...]
        cdm1 = cdvec - 1
        ii16 = lax.iota(jnp.int32, 16)
        skip16 = jnp.zeros((16,), jnp.int32) + skip
        lim16 = jnp.zeros((16,), jnp.int32) + (jnp.int32(E) - start)

        def _grp(i, cnt):
            o = pl.multiple_of(i * 16, 16)
            s16 = src_v[pl.ds(o, 16)]
            d16 = dst_v[pl.ds(o, 16)]
            t16 = typ_v[pl.ds(o, 16)]
            sd = plsc.load_gather(dist_v, [s16])
            dd = plsc.load_gather(dist_v, [d16])
            pos16 = o + ii16
            m = ((sd == cdvec) & (dd == cdm1)
                 & (pos16 >= skip16) & (pos16 < lim16))
            mi = m.astype(jnp.int32)
            pk = lax.bitwise_or(lax.shift_left(t16 * N + s16, SHIFT), d16)
            pos = HDR + cnt + plsc.cumsum(mi) - mi
            plsc.store_scatter(pk_v, [pos], pk, mask=m)
            return cnt + jnp.sum(mi)

        cnt = lax.fori_loop(0, EPW // 16, _grp, jnp.int32(0))

        dump16 = jnp.full((16,), N, jnp.int32)

        def _pad(k, carry):
            pos = HDR + cnt + k * 16 + ii16
            plsc.store_scatter(pk_v, [pos], dump16)
            return carry
        lax.fori_loop(0, CHUNK // 16, _pad, 0)
        pk_v[pl.ds(0, 16)] = cnt + jnp.zeros((16,), jnp.int32)

        pltpu.sync_copy(pk_v, pk_hbm.at[pl.ds(g * STR, STR)])

    return flt


def _make_gather(N, R, H):
    SPS = NC * NS // NS
    U = 32
    SLOTS = CHUNK // U
    ACCR = N + 112
    ZST = ACCR // NS
    mesh = plsc.VectorSubcoreMesh(
        core_axis_name="c", subcore_axis_name="s", num_cores=NC, num_subcores=NS)

    @functools.partial(
        pl.kernel,
        out_type=jax.ShapeDtypeStruct((NC, ACCR, H), jnp.float32),
        mesh=mesh,
        compiler_params=pltpu.CompilerParams(needs_layout_passes=False),
        scratch_types=[
            pltpu.VMEM((SPS * STR,), jnp.int32),
            pltpu.VMEM((CHUNK,), jnp.int32),
            pltpu.VMEM((CHUNK,), jnp.int32),
            pltpu.VMEM((CHUNK, H), jnp.float32),
            pltpu.VMEM_SHARED((N + 112, H), jnp.float32),
            pltpu.SemaphoreType.DMA,
            pltpu.SemaphoreType.DMA,
            pltpu.SemaphoreType.DMA,
            pltpu.SemaphoreType.DMA,
        ],
    )
    def gat(pk_hbm, z_hbm, nm_hbm,
            pk_v, uzi_v, udi_v, rows_v, acc_sh, gs0, gs1, gs2, gs3):
        gsems = (gs0, gs1, gs2, gs3)
        c = lax.axis_index("c")
        s = lax.axis_index("s")
        sbase = pl.multiple_of(s * (SPS * STR), SPS * STR)
        pltpu.async_copy(pk_hbm.at[pl.ds(sbase, SPS * STR)], pk_v, gs3)

        zero16f = jnp.zeros((16,), jnp.float32)

        def _zr(i, carry):
            for k in range(H // 16):
                rows_v[i, pl.ds(k * 16, 16)] = zero16f
            return carry
        lax.fori_loop(0, CHUNK, _zr, 0)

        zbase = pl.multiple_of(s * ZST, ZST)
        for off in range(0, ZST, CHUNK):
            n = min(CHUNK, ZST - off)
            pltpu.sync_copy(rows_v.at[pl.ds(0, n)],
                            acc_sh.at[pl.ds(zbase + off, n)])

        pltpu.make_async_copy(pk_hbm.at[pl.ds(sbase, SPS * STR)],
                              pk_v, gs3).wait()
        plsc.subcore_barrier()

        zoff = c * (R * N)
        ii16 = lax.iota(jnp.int32, 16)

        for h in range(SPS):
            cnt = pk_v[pl.ds(h * STR, 16)][0]
            nchunks = lax.div(cnt + (CHUNK - 1), jnp.int32(CHUNK))

            def _chunk(j, carry, _b=h * STR + HDR):
                base = _b + j * CHUNK
                for k in range(CHUNK // 16):
                    pk = plsc.load_gather(pk_v, [base + k * 16 + ii16])
                    uzi_v[pl.ds(k * 16, 16)] = (
                        lax.shift_right_logical(pk, SHIFT) + zoff)
                    udi_v[pl.ds(k * 16, 16)] = lax.bitwise_and(
                        pk, (1 << SHIFT) - 1)
                for b in range(SLOTS):
                    pltpu.async_copy(z_hbm.at[uzi_v.at[pl.ds(b * U, U)]],
                                     rows_v.at[pl.ds(b * U, U)], gsems[b])
                for b in range(SLOTS):
                    pltpu.make_async_copy(z_hbm.at[uzi_v.at[pl.ds(b * U, U)]],
                                          rows_v.at[pl.ds(b * U, U)],
                                          gsems[b]).wait()
                pltpu.sync_copy(rows_v, acc_sh.at[udi_v], add=True)
                return carry
            lax.fori_loop(0, nchunks, _chunk, 0)

        plsc.subcore_barrier()
        pltpu.sync_copy(acc_sh.at[pl.ds(zbase, ZST)],
                        nm_hbm.at[c, pl.ds(zbase, ZST)])

    return gat


def kernel(x, edge_index, edge_types, distances, current_distance,
           relation_weights, self_weight, bias):
    N, D = x.shape
    O = self_weight.shape[1]
    R = relation_weights.shape[0]
    E = edge_types.shape[0]
    H = O // 2

    cd = jnp.asarray(current_distance, dtype=jnp.int32)
    cd16 = jnp.full((16,), cd, dtype=jnp.int32)
    cd11 = cd.reshape(1, 1)
    dist2d = distances.reshape(N, 1)
    bias2d = bias.reshape(1, O)

    packed = _make_filter(N, E)(
        cd16, edge_index[0], edge_index[1], edge_types, distances)
    z = _z_table(x, relation_weights, N, D, R, H)
    nm = _make_gather(N, R, H)(packed, z)
    return _epilogue(cd11, x, dist2d, nm, self_weight, bias2d, N, D, O, H)


# speedup vs baseline: 1.1711x; 1.1711x over previous
import functools

import jax
import jax.numpy as jnp
from jax import lax
from jax.experimental import pallas as pl
from jax.experimental.pallas import tpu as pltpu
from jax.experimental.pallas import tpu_sc as plsc

NS = 16
NC = 2
CHUNK = 128
SEC = 2000
BN = 1000
SHIFT = 14


def _z_body(x_ref, w_ref, o_ref):
    o_ref[...] = jnp.dot(x_ref[...], w_ref[0], preferred_element_type=jnp.float32)


def _z_table(x, rw, N, D, R, H):
    nb = N // BN
    return pl.pallas_call(
        _z_body,
        grid=(R, NC, nb),
        in_specs=[
            pl.BlockSpec((BN, D), lambda r, c, n: (n, 0)),
            pl.BlockSpec((1, D, H), lambda r, c, n: (r, 0, c)),
        ],
        out_specs=pl.BlockSpec(
            (BN, H), lambda r, c, n, _nb=nb, _R=R: (c * _R * _nb + r * _nb + n, 0)),
        out_shape=jax.ShapeDtypeStruct((NC * R * N, H), jnp.float32),
    )(x, rw)


def _ep_body(cd_ref, x_ref, d_ref, nm0_ref, nm1_ref, ws_ref, b_ref, o_ref):
    cd = cd_ref[0, 0]
    tm = (d_ref[...] == cd - 1).astype(jnp.float32)
    nm = jnp.concatenate([nm0_ref[0], nm1_ref[0]], axis=-1) * tm
    acc = x_ref[...] + jnp.dot(x_ref[...], ws_ref[...],
                               preferred_element_type=jnp.float32)
    o_ref[...] = jnp.maximum(acc + b_ref[...] + nm, 0.0)


def _epilogue(cd11, x, dist2d, nm, ws, bias2d, N, D, O, H):
    nb = N // BN
    return pl.pallas_call(
        _ep_body,
        grid=(nb,),
        in_specs=[
            pl.BlockSpec((1, 1), lambda n: (0, 0)),
            pl.BlockSpec((BN, D), lambda n: (n, 0)),
            pl.BlockSpec((BN, 1), lambda n: (n, 0)),
            pl.BlockSpec((1, BN, H), lambda n: (0, n, 0)),
            pl.BlockSpec((1, BN, H), lambda n: (1, n, 0)),
            pl.BlockSpec((D, O), lambda n: (0, 0)),
            pl.BlockSpec((1, O), lambda n: (0, 0)),
        ],
        out_specs=pl.BlockSpec((BN, O), lambda n: (n, 0)),
        out_shape=jax.ShapeDtypeStruct((N, O), jnp.float32),
    )(cd11, x, dist2d, nm, nm, ws, bias2d)


def _make_sc(N, E, R, H):
    EPT = E // NS
    NSEC = EPT // SEC
    U = 32
    SLOTS = CHUNK // U
    CAP = EPT + 2 * CHUNK
    ACCR = N + 112
    ZST = ACCR // NS
    mesh = plsc.VectorSubcoreMesh(
        core_axis_name="c", subcore_axis_name="s", num_cores=NC, num_subcores=NS)

    @functools.partial(
        pl.kernel,
        out_type=jax.ShapeDtypeStruct((NC, ACCR, H), jnp.float32),
        mesh=mesh,
        compiler_params=pltpu.CompilerParams(needs_layout_passes=False),
        scratch_types=[
            pltpu.VMEM((16,), jnp.int32),
            pltpu.VMEM((N,), jnp.int32),
            pltpu.VMEM((SEC,), jnp.int32),
            pltpu.VMEM((SEC,), jnp.int32),
            pltpu.VMEM((SEC,), jnp.int32),
            pltpu.VMEM((CAP,), jnp.int32),
            pltpu.VMEM((CHUNK,), jnp.int32),
            pltpu.VMEM((CHUNK,), jnp.int32),
            pltpu.VMEM((CHUNK, H), jnp.float32),
            pltpu.VMEM_SHARED((N + 112, H), jnp.float32),
            pltpu.SemaphoreType.DMA,
            pltpu.SemaphoreType.DMA,
            pltpu.SemaphoreType.DMA,
            pltpu.SemaphoreType.DMA,
        ],
    )
    def sc(cd_hbm, src_hbm, dst_hbm, typ_hbm, dist_hbm, z_hbm, nm_hbm,
           cd_v, dist_v, src_v, dst_v, typ_v,
           pk_v, uzi_v, udi_v, rows_v, acc_sh, gs0, gs1, gs2, gs3):
        gsems = (gs0, gs1, gs2, gs3)
        c = lax.axis_index("c")
        s = lax.axis_index("s")
        ebase = pl.multiple_of(s * EPT, EPT)
        pltpu.sync_copy(cd_hbm, cd_v)
        pltpu.sync_copy(dist_hbm, dist_v)

        zero16f = jnp.zeros((16,), jnp.float32)

        def _zr(i, carry):
            for k in range(H // 16):
                rows_v[i, pl.ds(k * 16, 16)] = zero16f
            return carry
        lax.fori_loop(0, CHUNK, _zr, 0)

        zbase = pl.multiple_of(s * ZST, ZST)
        for off in range(0, ZST, CHUNK):
            n = min(CHUNK, ZST - off)
            pltpu.async_copy(rows_v.at[pl.ds(0, n)],
                             acc_sh.at[pl.ds(zbase + off, n)], gs3)

        cdvec = cd_v[...]
        cdm1 = cdvec - 1
        ii16 = lax.iota(jnp.int32, 16)

        def _grp(i, cnt):
            o = pl.multiple_of(i * 16, 16)
            s16 = src_v[pl.ds(o, 16)]
            d16 = dst_v[pl.ds(o, 16)]
            t16 = typ_v[pl.ds(o, 16)]
            sd = plsc.load_gather(dist_v, [s16])
            dd = plsc.load_gather(dist_v, [d16])
            m = (sd == cdvec) & (dd == cdm1)
            mi = m.astype(jnp.int32)
            pk = lax.bitwise_or(lax.shift_left(t16 * N + s16, SHIFT), d16)
            pos = cnt + plsc.cumsum(mi) - mi
            plsc.store_scatter(pk_v, [pos], pk, mask=m)
            return cnt + jnp.sum(mi)

        cnt = jnp.int32(0)
        for t in range(NSEC):
            sb = pl.multiple_of(ebase + t * SEC, SEC)
            pltpu.async_copy(src_hbm.at[pl.ds(sb, SEC)], src_v, gs0)
            pltpu.async_copy(dst_hbm.at[pl.ds(sb, SEC)], dst_v, gs1)
            pltpu.async_copy(typ_hbm.at[pl.ds(sb, SEC)], typ_v, gs2)
            pltpu.make_async_copy(src_hbm.at[pl.ds(sb, SEC)], src_v, gs0).wait()
            pltpu.make_async_copy(dst_hbm.at[pl.ds(sb, SEC)], dst_v, gs1).wait()
            pltpu.make_async_copy(typ_hbm.at[pl.ds(sb, SEC)], typ_v, gs2).wait()
            cnt = lax.fori_loop(0, SEC // 16, _grp, cnt)

        dump16 = jnp.full((16,), N, jnp.int32)

        def _pad(k, carry):
            pos = cnt + k * 16 + ii16
            plsc.store_scatter(pk_v, [pos], dump16)
            return carry
        lax.fori_loop(0, CHUNK // 16, _pad, 0)

        for off in range(0, ZST, CHUNK):
            n = min(CHUNK, ZST - off)
            pltpu.make_async_copy(rows_v.at[pl.ds(0, n)],
                                  acc_sh.at[pl.ds(zbase + off, n)], gs3).wait()
        plsc.subcore_barrier()

        zoff = c * (R * N)
        nchunks = lax.div(cnt + (CHUNK - 1), jnp.int32(CHUNK))

        def _chunk(j, carry):
            base = j * CHUNK
            for k in range(CHUNK // 16):
                pk = plsc.load_gather(pk_v, [base + k * 16 + ii16])
                uzi_v[pl.ds(k * 16, 16)] = (
                    lax.shift_right_logical(pk, SHIFT) + zoff)
                udi_v[pl.ds(k * 16, 16)] = lax.bitwise_and(
                    pk, (1 << SHIFT) - 1)
            for b in range(SLOTS):
                pltpu.async_copy(z_hbm.at[uzi_v.at[pl.ds(b * U, U)]],
                                 rows_v.at[pl.ds(b * U, U)], gsems[b])
            for b in range(SLOTS):
                pltpu.make_async_copy(z_hbm.at[uzi_v.at[pl.ds(b * U, U)]],
                                      rows_v.at[pl.ds(b * U, U)],
                                      gsems[b]).wait()
            pltpu.sync_copy(rows_v, acc_sh.at[udi_v], add=True)
            return carry
        lax.fori_loop(0, nchunks, _chunk, 0)

        plsc.subcore_barrier()
        pltpu.sync_copy(acc_sh.at[pl.ds(zbase, ZST)],
                        nm_hbm.at[c, pl.ds(zbase, ZST)])

    return sc


def kernel(x, edge_index, edge_types, distances, current_distance,
           relation_weights, self_weight, bias):
    N, D = x.shape
    O = self_weight.shape[1]
    R = relation_weights.shape[0]
    E = edge_types.shape[0]
    H = O // 2

    cd = jnp.asarray(current_distance, dtype=jnp.int32)
    cd16 = jnp.full((16,), cd, dtype=jnp.int32)
    cd11 = cd.reshape(1, 1)
    dist2d = distances.reshape(N, 1)
    bias2d = bias.reshape(1, O)

    z = _z_table(x, relation_weights, N, D, R, H)
    nm = _make_sc(N, E, R, H)(
        cd16, edge_index[0], edge_index[1], edge_types, distances, z)
    return _epilogue(cd11, x, dist2d, nm, self_weight, bias2d, N, D, O, H)
